# Initial kernel scaffold; baseline (speedup 1.0000x reference)
import jax
import jax.numpy as jnp
from jax.experimental import pallas as pl


def kernel(x_domain, x_slot, x_value, edge_dd, edge_ds, edge_sv, Wt_dom, bt_dom, Ws_dom, bs_dom, Wt_slot, bt_slot, Ws_slot, bs_slot, Wt_val, bt_val, Ws_val, bs_val, g_dom, be_dom, g_slot, be_slot, g_val, be_val):
    raise NotImplementedError("write your pallas kernel here")


# speedup vs baseline: 1.5614x; 1.5614x over previous
import functools

import jax
import jax.numpy as jnp
from jax import lax
from jax.experimental import pallas as pl
from jax.experimental.pallas import tpu as pltpu
from jax.experimental.pallas import tpu_sc as plsc

N = 10000
D = 256
E = 160000
NC = 2
NS = 16
HALF = D // 2
EPS = E // NS
K = 40
NCHUNK = EPS // K
NP = 10240
ROWS = NP // NS


def _mm_body(xs_ref, ws_ref, bs_ref, xt_ref, wt_ref, bt_ref, slab_out, xt_out):
    ys = jnp.dot(xs_ref[0], ws_ref[0], preferred_element_type=jnp.float32) + bs_ref[0]
    yt = jnp.dot(xt_ref[0], wt_ref[0], preferred_element_type=jnp.float32) + bt_ref[0]
    slab_out[0, 0] = ys[:, :HALF]
    slab_out[0, 1] = ys[:, HALF:]
    xt_out[0] = yt


def _mm_call(xsrc, Ws, bs, xtgt, Wt, bt):
    R = 1000
    nb = N // R
    return pl.pallas_call(
        _mm_body,
        grid=(3, nb),
        in_specs=[
            pl.BlockSpec((1, R, D), lambda r, i: (r, i, 0)),
            pl.BlockSpec((1, D, D), lambda r, i: (r, 0, 0)),
            pl.BlockSpec((1, 1, D), lambda r, i: (r, 0, 0)),
            pl.BlockSpec((1, R, D), lambda r, i: (r, i, 0)),
            pl.BlockSpec((1, D, D), lambda r, i: (r, 0, 0)),
            pl.BlockSpec((1, 1, D), lambda r, i: (r, 0, 0)),
        ],
        out_specs=[
            pl.BlockSpec((1, 2, R, HALF), lambda r, i: (r, 0, i, 0)),
            pl.BlockSpec((1, R, D), lambda r, i: (r, i, 0)),
        ],
        out_shape=[
            jax.ShapeDtypeStruct((3, NC, N, HALF), jnp.float32),
            jax.ShapeDtypeStruct((3, N, D), jnp.float32),
        ],
    )(xsrc, Ws, bs, xtgt, Wt, bt)


@functools.lru_cache(maxsize=None)
def _make_sc_scatter():
    mesh = plsc.VectorSubcoreMesh(
        core_axis_name="c", subcore_axis_name="s", num_cores=NC, num_subcores=NS
    )
    return pl.kernel(
        _sc_scatter_body,
        out_type=(
            jax.ShapeDtypeStruct((3 * NC * NP, HALF), jnp.float32),
            jax.ShapeDtypeStruct((3 * NC * NP, HALF), jnp.float32),
        ),
        mesh=mesh,
        scratch_types=[
            pltpu.VMEM_SHARED((NP, HALF), jnp.float32),
            pltpu.VMEM((K,), jnp.int32),
            pltpu.VMEM((K,), jnp.int32),
            pltpu.VMEM((K,), jnp.int32),
            pltpu.VMEM((K, HALF), jnp.float32),
            pltpu.VMEM((K, HALF), jnp.float32),
            pltpu.SemaphoreType.DMA,
        ],
    )


def _sc_scatter_body(xs_rows, srcf, dst_all, zidx_hbm, acc_out, deg_out,
                     acc_sh, src_buf, dst_buf, zidx_buf, rows_buf,
                     ones_buf, sem):
    c = lax.axis_index("c")
    s = lax.axis_index("s")
    zeros16 = jnp.zeros((16,), jnp.float32)
    ones16 = jnp.ones((16,), jnp.float32)
    iota16 = lax.iota(jnp.int32, 16)

    def fill_const(i, carry):
        for j in range(HALF // 16):
            ones_buf[i, pl.ds(j * 16, 16)] = ones16
        return carry

    lax.fori_loop(0, K, fill_const, 0)

    base = s * ROWS

    def set_zidx(off):
        pltpu.sync_copy(zidx_hbm.at[pl.ds(off, K)], zidx_buf)

    for r in range(3):
        def fill_zrows(i, carry):
            for j in range(HALF // 16):
                rows_buf[i, pl.ds(j * 16, 16)] = zeros16
            return carry

        lax.fori_loop(0, K, fill_zrows, 0)

        def zero_acc():
            for z in range(ROWS // K):
                pltpu.sync_copy(zidx_hbm.at[pl.ds(base + z * K, K)], zidx_buf)
                pltpu.sync_copy(rows_buf, acc_sh.at[zidx_buf])

        zero_acc()
        plsc.subcore_barrier()

        def chunk_body(i, carry):
            soff = ((r * NC + c) * NS + s) * EPS + i * K
            doff = r * E + s * EPS + i * K
            pltpu.sync_copy(srcf.at[pl.ds(soff, K)], src_buf)
            pltpu.sync_copy(dst_all.at[pl.ds(doff, K)], dst_buf)
            pltpu.async_copy(xs_rows.at[src_buf], rows_buf, sem).wait()
            pltpu.sync_copy(rows_buf, acc_sh.at[dst_buf], add=True)
            return carry

        lax.fori_loop(0, NCHUNK, chunk_body, 0)
        plsc.subcore_barrier()

        aoff0 = (r * NC + c) * NP + base
        for z in range(ROWS // K):
            pltpu.sync_copy(zidx_hbm.at[pl.ds(base + z * K, K)], zidx_buf)
            pltpu.sync_copy(acc_sh.at[zidx_buf], rows_buf)
            pltpu.sync_copy(rows_buf, acc_out.at[pl.ds(aoff0 + z * K, K)])
        plsc.subcore_barrier()

        def fill_z2(i, carry):
            for j in range(HALF // 16):
                rows_buf[i, pl.ds(j * 16, 16)] = zeros16
            return carry

        lax.fori_loop(0, K, fill_z2, 0)
        zero_acc()
        plsc.subcore_barrier()

        def deg_body(i, carry):
            doff = r * E + c * (E // NC) + s * (E // NC // NS) + i * K
            pltpu.sync_copy(dst_all.at[pl.ds(doff, K)], dst_buf)
            pltpu.sync_copy(ones_buf, acc_sh.at[dst_buf], add=True)
            return carry

        lax.fori_loop(0, E // NC // NS // K, deg_body, 0)
        plsc.subcore_barrier()

        for z in range(ROWS // K):
            pltpu.sync_copy(zidx_hbm.at[pl.ds(base + z * K, K)], zidx_buf)
            pltpu.sync_copy(acc_sh.at[zidx_buf], rows_buf)
            pltpu.sync_copy(rows_buf, deg_out.at[pl.ds(aoff0 + z * K, K)])
        plsc.subcore_barrier()


def _fin_body(acc_ref, deg_ref, xt_ref, xr_ref, g_ref, b_ref, o_ref):
    deg = deg_ref[0, 0, :, 0:1] + deg_ref[0, 1, :, 0:1]
    deg = jnp.maximum(deg, 1.0)
    accf = jnp.concatenate([acc_ref[0, 0], acc_ref[0, 1]], axis=1)
    h = accf / deg + xt_ref[0] + xr_ref[0]
    m = jnp.mean(h, axis=-1, keepdims=True)
    v = jnp.mean((h - m) ** 2, axis=-1, keepdims=True)
    y = (h - m) * lax.rsqrt(v + 1e-5) * g_ref[0] + b_ref[0]
    o_ref[0] = jnp.maximum(y, 0.0)


def _fin_call(acc, degp, xt, xres, g, be):
    R = 1000
    nb = N // R
    return pl.pallas_call(
        _fin_body,
        grid=(3, nb),
        in_specs=[
            pl.BlockSpec((1, NC, R, HALF), lambda r, i: (r, 0, i, 0)),
            pl.BlockSpec((1, NC, R, HALF), lambda r, i: (r, 0, i, 0)),
            pl.BlockSpec((1, R, D), lambda r, i: (r, i, 0)),
            pl.BlockSpec((1, R, D), lambda r, i: (r, i, 0)),
            pl.BlockSpec((1, 1, D), lambda r, i: (r, 0, 0)),
            pl.BlockSpec((1, 1, D), lambda r, i: (r, 0, 0)),
        ],
        out_specs=pl.BlockSpec((1, R, D), lambda r, i: (r, i, 0)),
        out_shape=jax.ShapeDtypeStruct((3, N, D), jnp.float32),
    )(acc, degp, xt, xres, g, be)


def kernel(x_domain, x_slot, x_value, edge_dd, edge_ds, edge_sv,
           Wt_dom, bt_dom, Ws_dom, bs_dom,
           Wt_slot, bt_slot, Ws_slot, bs_slot,
           Wt_val, bt_val, Ws_val, bs_val,
           g_dom, be_dom, g_slot, be_slot, g_val, be_val):
    xsrc = jnp.stack([x_domain, x_domain, x_slot])
    xtgt = jnp.stack([x_domain, x_slot, x_value])
    Wt = jnp.stack([Wt_dom, Wt_slot, Wt_val])
    bt = jnp.stack([bt_dom, bt_slot, bt_val])[:, None, :]
    Ws = jnp.stack([Ws_dom, Ws_slot, Ws_val])
    bs = jnp.stack([bs_dom, bs_slot, bs_val])[:, None, :]
    g = jnp.stack([g_dom, g_slot, g_val])[:, None, :]
    be = jnp.stack([be_dom, be_slot, be_val])[:, None, :]

    src = jnp.stack([edge_dd[0], edge_ds[0], edge_sv[0]]).astype(jnp.int32)
    dst = jnp.stack([edge_dd[1], edge_ds[1], edge_sv[1]]).astype(jnp.int32)
    offs = (jnp.arange(3, dtype=jnp.int32)[:, None] * NC
            + jnp.arange(NC, dtype=jnp.int32)[None, :]) * N
    srcf = (src[:, None, :] + offs[:, :, None]).reshape(3 * NC * E)
    dst = dst.reshape(3 * E)

    xs_slab, xt_all = _mm_call(xsrc, Ws, bs, xtgt, Wt, bt)
    xs_rows = xs_slab.reshape(3 * NC * N, HALF)
    zidx = jnp.arange(NP, dtype=jnp.int32)
    acc, degp = _make_sc_scatter()(xs_rows, srcf, dst, zidx)
    acc = acc.reshape(3, NC, NP, HALF)
    degp = degp.reshape(3, NC, NP, HALF)
    out = _fin_call(acc, degp, xt_all, xtgt, g, be)
    return (out[0], out[1], out[2])
